# baseline (speedup 1.0000x reference)
import jax
import jax.numpy as jnp
from jax import lax
from jax.experimental import pallas as pl
from jax.experimental.pallas import tpu as pltpu
from jax.experimental.pallas import tpu_sc as plsc

D = 512
EPS = 1e-5
NW = 32
CHUNK = 64


def _make_sc_gather(n_tok):
    tok_per_w = n_tok // NW
    n_chunks = tok_per_w // CHUNK
    mesh = plsc.VectorSubcoreMesh(core_axis_name="c", subcore_axis_name="s")

    def body(x_hbm, table_hbm, out_hbm, idx_v, b0, b1, g0, g1, o0):
        wid = lax.axis_index("s") * 2 + lax.axis_index("c")
        base = wid * tok_per_w
        pltpu.sync_copy(x_hbm.at[wid], idx_v)

        bufs = (b0, b1)
        gsems = (g0, g1)

        def gather(c):
            return pltpu.async_copy(
                table_hbm.at[idx_v.at[c]], bufs[c % 2], gsems[c % 2])

        gathers = [None] * n_chunks
        gathers[0] = gather(0)
        for c in range(n_chunks):
            if c + 1 < n_chunks:
                gathers[c + 1] = gather(c + 1)
            gathers[c].wait()
            pltpu.async_copy(
                bufs[c % 2], out_hbm.at[pl.ds(base + c * CHUNK, CHUNK)],
                o0).wait()

    return pl.kernel(
        body,
        out_type=jax.ShapeDtypeStruct((n_tok, D), jnp.float32),
        mesh=mesh,
        scratch_types=[
            pltpu.VMEM((n_chunks, CHUNK), jnp.int32),
            pltpu.VMEM((CHUNK, D), jnp.float32),
            pltpu.VMEM((CHUNK, D), jnp.float32),
            pltpu.SemaphoreType.DMA,
            pltpu.SemaphoreType.DMA,
            pltpu.SemaphoreType.DMA,
        ],
    )


def _tc_ln_body(rows_ref, pos_ref, mod_ref, g_ref, b_ref, o_ref):
    h = rows_ref[...] + pos_ref[...][None] + mod_ref[...][None]
    mean = jnp.mean(h, axis=-1, keepdims=True)
    meansq = jnp.mean(h * h, axis=-1, keepdims=True)
    var = meansq - mean * mean
    scale = lax.rsqrt(var + EPS) * g_ref[...][None]
    shift = b_ref[...][None] - mean * scale
    o_ref[...] = h * scale + shift


def _tc_ln(rows3d, pos2d, mod_row, g2d, b2d, ts, bsz, seq):
    return pl.pallas_call(
        _tc_ln_body,
        grid=(seq // ts,),
        in_specs=[
            pl.BlockSpec((bsz, ts, D), lambda j: (0, j, 0)),
            pl.BlockSpec((ts, D), lambda j: (j, 0)),
            pl.BlockSpec((1, D), lambda j: (0, 0)),
            pl.BlockSpec((1, D), lambda j: (0, 0)),
            pl.BlockSpec((1, D), lambda j: (0, 0)),
        ],
        out_specs=pl.BlockSpec((bsz, ts, D), lambda j: (0, j, 0)),
        out_shape=jax.ShapeDtypeStruct((bsz, seq, D), jnp.float32),
    )(rows3d, pos2d, mod_row, g2d, b2d)


def kernel(x, token_table, pos_emb, mod_table, gamma, beta):
    bsz, seq = x.shape
    n_tok = bsz * seq
    n_chunks = n_tok // NW // CHUNK
    x_arr = x.astype(jnp.int32).reshape(NW, n_chunks, CHUNK)
    rows = _make_sc_gather(n_tok)(x_arr, token_table)
    pos2d = pos_emb.reshape(seq, D)
    return _tc_ln(rows.reshape(bsz, seq, D), pos2d, mod_table[0:1],
                  gamma.reshape(1, D), beta.reshape(1, D), 1024, bsz, seq)
